# baseline (speedup 1.0000x reference)
import functools

import jax
import jax.numpy as jnp
from jax import lax
from jax.experimental import pallas as pl
from jax.experimental.pallas import tpu as pltpu
from jax.experimental.pallas import tpu_sc as plsc

F32 = jnp.float32
NC = 2
NS = 16
NW = NC * NS


def _prep_body(nf, st, W_v, b_v, W_u, b_u, We1a, We1b, We1d, be1,
               v_o, A_o, B_o, u_o, c_o):
    v = jnp.maximum(jnp.dot(nf[...], W_v[...],
                            preferred_element_type=F32) + b_v[...], 0.0)
    v_o[...] = v
    A_o[...] = jnp.dot(v, We1a[...], preferred_element_type=F32)
    B_o[...] = jnp.dot(v, We1b[...], preferred_element_type=F32)
    u = jnp.maximum(jnp.dot(st[...], W_u[...],
                            preferred_element_type=F32) + b_u[...], 0.0)
    u_o[...] = u
    c_o[...] = jnp.dot(u, We1d[...], preferred_element_type=F32) + be1[...]


def _prep(nf, st, W_v, b_v, W_u, b_u, We1a, We1b, We1d, be1, BN):
    N, D = nf.shape
    grid = (N // BN,)
    row = lambda i: (i, 0)
    fix = lambda i: (0, 0)
    full = lambda i: (0, 0)
    return pl.pallas_call(
        _prep_body,
        grid=grid,
        in_specs=[
            pl.BlockSpec((BN, D), row),
            pl.BlockSpec((1, D), fix),
            pl.BlockSpec((D, D), full),
            pl.BlockSpec((1, D), fix),
            pl.BlockSpec((D, D), full),
            pl.BlockSpec((1, D), fix),
            pl.BlockSpec((D, D), full),
            pl.BlockSpec((D, D), full),
            pl.BlockSpec((D, D), full),
            pl.BlockSpec((1, D), fix),
        ],
        out_specs=[
            pl.BlockSpec((BN, D), row),
            pl.BlockSpec((BN, D), row),
            pl.BlockSpec((BN, D), row),
            pl.BlockSpec((1, D), fix),
            pl.BlockSpec((1, D), fix),
        ],
        out_shape=[
            jax.ShapeDtypeStruct((N, D), F32),
            jax.ShapeDtypeStruct((N, D), F32),
            jax.ShapeDtypeStruct((N, D), F32),
            jax.ShapeDtypeStruct((1, D), F32),
            jax.ShapeDtypeStruct((1, D), F32),
        ],
    )(nf, st, W_v, b_v, W_u, b_u, We1a, We1b, We1d, be1)


def _sc_gather(A, B, src, dst, CHUNK=128):
    N, D = A.shape
    E = src.shape[0]
    EPW = E // NW
    n_full = EPW // CHUNK
    tail = EPW - n_full * CHUNK
    mesh = plsc.VectorSubcoreMesh(core_axis_name="c", subcore_axis_name="s")

    def body(A_hbm, B_hbm, src_hbm, dst_hbm, G_hbm,
             idx_s, idx_d, bufA, bufB, idx_st, idx_dt, bufAt, bufBt, sem):
        wid = lax.axis_index("s") * NC + lax.axis_index("c")
        base_w = wid * EPW

        def do_chunk(base, n, i_s, i_d, bA, bB):
            pltpu.sync_copy(src_hbm.at[pl.ds(base, n)], i_s)
            pltpu.sync_copy(dst_hbm.at[pl.ds(base, n)], i_d)
            cpA = pltpu.async_copy(A_hbm.at[i_s], bA, sem)
            cpB = pltpu.async_copy(B_hbm.at[i_d], bB, sem)
            cpA.wait()
            cpB.wait()

            def add_row(r, _):
                for l in range(D // 16):
                    sl = pl.ds(l * 16, 16)
                    bA[r, sl] = bA[r, sl] + bB[r, sl]
                return 0

            lax.fori_loop(0, n, add_row, 0)
            pltpu.sync_copy(bA, G_hbm.at[pl.ds(base, n)])

        def chunk_loop(j, _):
            do_chunk(base_w + j * CHUNK, CHUNK, idx_s, idx_d, bufA, bufB)
            return 0

        lax.fori_loop(0, n_full, chunk_loop, 0)
        if tail:
            do_chunk(base_w + n_full * CHUNK, tail, idx_st, idx_dt, bufAt, bufBt)

    run = pl.kernel(
        body,
        out_type=jax.ShapeDtypeStruct((E, D), F32),
        mesh=mesh,
        scratch_types=[
            pltpu.VMEM((CHUNK,), jnp.int32),
            pltpu.VMEM((CHUNK,), jnp.int32),
            pltpu.VMEM((CHUNK, D), F32),
            pltpu.VMEM((CHUNK, D), F32),
            pltpu.VMEM((max(tail, 8),), jnp.int32),
            pltpu.VMEM((max(tail, 8),), jnp.int32),
            pltpu.VMEM((max(tail, 8), D), F32),
            pltpu.VMEM((max(tail, 8), D), F32),
            pltpu.SemaphoreType.DMA,
        ],
    )
    return run(A, B, src, dst)


def _edge_body(e0, G, W_e, b_e, We1c, c_row, We2, be2,
               oe_o, en_o, esum_o):
    i = pl.program_id(0)
    e = jnp.maximum(jnp.dot(e0[...], W_e[...],
                            preferred_element_type=F32) + b_e[...], 0.0)
    t = jnp.dot(e, We1c[...], preferred_element_type=F32) + G[...] + c_row[...]
    h = jnp.maximum(t, 0.0)
    en = jnp.maximum(jnp.dot(h, We2[...],
                             preferred_element_type=F32) + be2[...], 0.0)
    en_o[...] = en
    oe_o[...] = en + e0[...]

    @pl.when(i == 0)
    def _():
        esum_o[...] = jnp.zeros_like(esum_o)

    esum_o[...] += jnp.sum(en, axis=0, keepdims=True)


def _edge_pass(e0, G, W_e, b_e, We1c, c_row, We2, be2, BE, off=0, oe_in=None):
    E, D = e0.shape
    H = G.shape[0]
    grid = (H // BE,)
    obk = off // BE
    rowo = lambda i: (i + obk, 0)
    row = lambda i: (i, 0)
    fix = lambda i: (0, 0)
    in_specs = [
        pl.BlockSpec((BE, D), rowo),
        pl.BlockSpec((BE, D), row),
        pl.BlockSpec((D, D), fix),
        pl.BlockSpec((1, D), fix),
        pl.BlockSpec((D, D), fix),
        pl.BlockSpec((1, D), fix),
        pl.BlockSpec((D, D), fix),
        pl.BlockSpec((1, D), fix),
    ]
    args = [e0, G, W_e, b_e, We1c, c_row, We2, be2]
    aliases = {}
    if oe_in is not None:
        in_specs.append(pl.BlockSpec((BE, D), rowo))
        args.append(oe_in)
        aliases = {8: 0}

    def body(*refs):
        if oe_in is not None:
            e0r, Gr, W_er, b_er, We1cr, c_rowr, We2r, be2r, _oe, oo, eo, so = refs
        else:
            e0r, Gr, W_er, b_er, We1cr, c_rowr, We2r, be2r, oo, eo, so = refs
        _edge_body(e0r, Gr, W_er, b_er, We1cr, c_rowr, We2r, be2r, oo, eo, so)

    return pl.pallas_call(
        body,
        grid=grid,
        in_specs=in_specs,
        out_specs=[
            pl.BlockSpec((BE, D), rowo),
            pl.BlockSpec((BE, D), row),
            pl.BlockSpec((1, D), fix),
        ],
        out_shape=[
            jax.ShapeDtypeStruct((E, D), F32),
            jax.ShapeDtypeStruct((H, D), F32),
            jax.ShapeDtypeStruct((1, D), F32),
        ],
        input_output_aliases=aliases,
    )(*args)


def _sc_common(NP, CHUNK):
    NPS = NP // NS
    n_init = NPS // CHUNK
    init_tail = NPS - n_init * CHUNK
    return NPS, n_init, init_tail


def _mk_helpers(D, CHUNK, NPS, n_init, init_tail, sid, cid, NP):
    def fill(b, val):
        def fr(i, _):
            for l in range(D // 16):
                b[i, pl.ds(l * 16, 16)] = jnp.full((16,), val, F32)
            return 0
        lax.fori_loop(0, CHUNK, fr, 0)

    off0 = sid * NPS + n_init * CHUNK

    def zero_spmem(shS, buf2):
        fill(buf2, 0.0)

        def init_blk(k, _):
            pltpu.sync_copy(buf2,
                            shS.at[pl.ds(sid * NPS + k * CHUNK, CHUNK)])
            return 0
        lax.fori_loop(0, n_init, init_blk, 0)
        if init_tail:
            pltpu.sync_copy(buf2.at[pl.ds(0, init_tail)],
                            shS.at[pl.ds(off0, init_tail)])

    def writeout(shS, buf2, out_ref):
        def out_blk(k, _):
            off = sid * NPS + k * CHUNK
            pltpu.sync_copy(shS.at[pl.ds(off, CHUNK)], buf2)
            pltpu.sync_copy(buf2, out_ref.at[pl.ds(cid * NP + off, CHUNK)])
            return 0
        lax.fori_loop(0, n_init, out_blk, 0)
        if init_tail:
            pltpu.sync_copy(shS.at[pl.ds(off0, init_tail)],
                            buf2.at[pl.ds(0, init_tail)])
            pltpu.sync_copy(buf2.at[pl.ds(0, init_tail)],
                            out_ref.at[pl.ds(cid * NP + off0, init_tail)])

    return fill, zero_spmem, writeout


def _sc_sums(e_new, dst, NP, CHUNK=80):
    E, D = e_new.shape
    EPW = E // NW
    n_full = EPW // CHUNK
    tail = EPW - n_full * CHUNK
    TB = max(tail, 8)
    NPS, n_init, init_tail = _sc_common(NP, CHUNK)
    mesh = plsc.VectorSubcoreMesh(core_axis_name="c", subcore_axis_name="s")

    def body(en_hbm, dst_hbm, S_hbm, shS, buf, buf2, idx, buf_t, idx_t, sem):
        cid = lax.axis_index("c")
        sid = lax.axis_index("s")
        wid = sid * NC + cid
        base_w = wid * EPW
        fill, zero_spmem, writeout = _mk_helpers(
            D, CHUNK, NPS, n_init, init_tail, sid, cid, NP)

        zero_spmem(shS, buf2)
        plsc.subcore_barrier()

        def chunkA(j, _):
            base = base_w + j * CHUNK
            pltpu.sync_copy(dst_hbm.at[pl.ds(base, CHUNK)], idx)
            pltpu.sync_copy(en_hbm.at[pl.ds(base, CHUNK)], buf)
            pltpu.sync_copy(buf, shS.at[idx], add=True)
            return 0
        lax.fori_loop(0, n_full, chunkA, 0)
        if tail:
            base = base_w + n_full * CHUNK
            pltpu.sync_copy(dst_hbm.at[pl.ds(base, tail)], idx_t)
            pltpu.sync_copy(en_hbm.at[pl.ds(base, tail)], buf_t)
            pltpu.sync_copy(buf_t, shS.at[idx_t], add=True)
        plsc.subcore_barrier()
        writeout(shS, buf2, S_hbm)

    run = pl.kernel(
        body,
        out_type=jax.ShapeDtypeStruct((NC * NP, D), F32),
        mesh=mesh,
        scratch_types=[
            pltpu.VMEM_SHARED((NP, D), F32),
            pltpu.VMEM((CHUNK, D), F32),
            pltpu.VMEM((CHUNK, D), F32),
            pltpu.VMEM((CHUNK,), jnp.int32),
            pltpu.VMEM((TB, D), F32),
            pltpu.VMEM((TB,), jnp.int32),
            pltpu.SemaphoreType.DMA,
        ],
    )
    return run(e_new, dst)


def _sc_counts(dst, E, NP, CHUNK=80):
    D = 128
    EPW = E // NW
    n_full = EPW // CHUNK
    NPS, n_init, init_tail = _sc_common(NP, CHUNK)
    mesh = plsc.VectorSubcoreMesh(core_axis_name="c", subcore_axis_name="s")

    def body(dst_hbm, C_hbm, shS, buf, buf2, idx, sem):
        cid = lax.axis_index("c")
        sid = lax.axis_index("s")
        wid = sid * NC + cid
        base_w = wid * EPW
        fill, zero_spmem, writeout = _mk_helpers(
            D, CHUNK, NPS, n_init, init_tail, sid, cid, NP)

        zero_spmem(shS, buf2)
        fill(buf, 1.0)
        plsc.subcore_barrier()

        def chunkB(j, _):
            base = base_w + j * CHUNK
            pltpu.sync_copy(dst_hbm.at[pl.ds(base, CHUNK)], idx)
            pltpu.sync_copy(buf, shS.at[idx], add=True)
            return 0
        lax.fori_loop(0, n_full, chunkB, 0)
        plsc.subcore_barrier()
        writeout(shS, buf2, C_hbm)

    run = pl.kernel(
        body,
        out_type=jax.ShapeDtypeStruct((NC * NP, D), F32),
        mesh=mesh,
        scratch_types=[
            pltpu.VMEM_SHARED((NP, D), F32),
            pltpu.VMEM((CHUNK, D), F32),
            pltpu.VMEM((CHUNK, D), F32),
            pltpu.VMEM((CHUNK,), jnp.int32),
            pltpu.SemaphoreType.DMA,
        ],
    )
    return run(dst)


def _finish_body(S, C16, v, nf, u, st, esum,
                 Wn1a, Wn1b, Wn1c, bn1, Wn2, bn2,
                 Ws1a, Ws1b, Ws1c, bs1, Ws2, bs2,
                 inv_e, inv_n,
                 ov_o, ou_o, vsum):
    i = pl.program_id(0)
    n_blocks = pl.num_programs(0)
    s = jnp.sum(S[...], axis=0)
    cnt2 = C16[0] + C16[1]
    cnt = cnt2[:, 0:1]
    ve = s / jnp.maximum(cnt, 1.0)
    pre = (jnp.dot(v[...], Wn1a[...], preferred_element_type=F32)
           + jnp.dot(ve, Wn1b[...], preferred_element_type=F32)
           + jnp.dot(u[...], Wn1c[...], preferred_element_type=F32)
           + bn1[...])
    h = jnp.maximum(pre, 0.0)
    vn = jnp.maximum(jnp.dot(h, Wn2[...],
                             preferred_element_type=F32) + bn2[...], 0.0)
    ov_o[...] = vn + nf[...]

    @pl.when(i == 0)
    def _():
        vsum[...] = jnp.zeros_like(vsum)

    vsum[...] += jnp.sum(vn, axis=0, keepdims=True)

    @pl.when(i == n_blocks - 1)
    def _():
        u_edge = esum[...] * inv_e[0]
        u_vertex = vsum[...] * inv_n[0]
        spre = (jnp.dot(u[...], Ws1a[...], preferred_element_type=F32)
                + jnp.dot(u_edge, Ws1b[...], preferred_element_type=F32)
                + jnp.dot(u_vertex, Ws1c[...], preferred_element_type=F32)
                + bs1[...])
        h2 = jnp.maximum(spre, 0.0)
        un = jnp.maximum(jnp.dot(h2, Ws2[...],
                                 preferred_element_type=F32) + bs2[...], 0.0)
        ou_o[...] = un + st[...]


def _finish(S, C16, v, nf, u, st, esum,
            Wn1a, Wn1b, Wn1c, bn1, Wn2, bn2,
            Ws1a, Ws1b, Ws1c, bs1, Ws2, bs2, E, BN):
    N, D = nf.shape
    KS = S.shape[0]
    CW = C16.shape[-1]
    grid = (N // BN,)
    row = lambda i: (i, 0)
    row3 = lambda i: (0, i, 0)
    fix = lambda i: (0, 0)
    inv_e = jnp.full((1,), 1.0 / E, F32)
    inv_n = jnp.full((1,), 1.0 / N, F32)
    return pl.pallas_call(
        _finish_body,
        grid=grid,
        in_specs=[
            pl.BlockSpec((KS, BN, D), row3),
            pl.BlockSpec((2, BN, CW), row3),
            pl.BlockSpec((BN, D), row),
            pl.BlockSpec((BN, D), row),
            pl.BlockSpec((1, D), fix),
            pl.BlockSpec((1, D), fix),
            pl.BlockSpec((1, D), fix),
            pl.BlockSpec((D, D), fix),
            pl.BlockSpec((D, D), fix),
            pl.BlockSpec((D, D), fix),
            pl.BlockSpec((1, D), fix),
            pl.BlockSpec((D, D), fix),
            pl.BlockSpec((1, D), fix),
            pl.BlockSpec((D, D), fix),
            pl.BlockSpec((D, D), fix),
            pl.BlockSpec((D, D), fix),
            pl.BlockSpec((1, D), fix),
            pl.BlockSpec((D, D), fix),
            pl.BlockSpec((1, D), fix),
            pl.BlockSpec(memory_space=pltpu.SMEM),
            pl.BlockSpec(memory_space=pltpu.SMEM),
        ],
        out_specs=[
            pl.BlockSpec((BN, D), row),
            pl.BlockSpec((1, D), fix),
        ],
        out_shape=[
            jax.ShapeDtypeStruct((N, D), F32),
            jax.ShapeDtypeStruct((1, D), F32),
        ],
        scratch_shapes=[pltpu.VMEM((1, D), F32)],
    )(S, C16, v, nf, u, st, esum,
      Wn1a, Wn1b, Wn1c, bn1, Wn2, bn2,
      Ws1a, Ws1b, Ws1c, bs1, Ws2, bs2, inv_e, inv_n)


def kernel(edge_feat, node_feat, state_attr, edge_index,
           W_e, b_e, W_v, b_v, W_u, b_u,
           We1, be1, We2, be2,
           Wn1, bn1, Wn2, bn2,
           Ws1, bs1, Ws2, bs2):
    E, D = edge_feat.shape
    N = node_feat.shape[0]
    src = edge_index[0]
    dst = edge_index[1]

    We1a, We1b = We1[0:D], We1[D:2 * D]
    We1c, We1d = We1[2 * D:3 * D], We1[3 * D:4 * D]
    Wn1a, Wn1b, Wn1c = Wn1[0:D], Wn1[D:2 * D], Wn1[2 * D:3 * D]
    Ws1a, Ws1b, Ws1c = Ws1[0:D], Ws1[D:2 * D], Ws1[2 * D:3 * D]
    r = lambda x: x.reshape(1, D)

    v, A, B, u, c_row = _prep(node_feat, state_attr, W_v, r(b_v), W_u, r(b_u),
                              We1a, We1b, We1d, r(be1), BN=1000)
    NP = ((N + NS * 8 - 1) // (NS * 8)) * (NS * 8)
    H = E // 2
    G1 = _sc_gather(A, B, src[:H], dst[:H])
    G2 = _sc_gather(A, B, src[H:], dst[H:])
    C = _sc_counts(dst, E, NP)
    oe1, en1, es1 = _edge_pass(edge_feat, G1, W_e, r(b_e), We1c, c_row,
                               We2, r(be2), BE=4000, off=0)
    out_e, en2, es2 = _edge_pass(edge_feat, G2, W_e, r(b_e), We1c, c_row,
                                 We2, r(be2), BE=4000, off=H, oe_in=oe1)
    S1 = _sc_sums(en1, dst[:H], NP)
    S2 = _sc_sums(en2, dst[H:], NP)
    e_sum = es1 + es2
    S = jnp.concatenate([S1, S2], axis=0).reshape(2 * NC, NP, D)
    C16 = C.reshape(NC, NP, D)
    out_v, out_u = _finish(S, C16, v, node_feat, u, state_attr, e_sum,
                           Wn1a, Wn1b, Wn1c, r(bn1), Wn2, r(bn2),
                           Ws1a, Ws1b, Ws1c, r(bs1), Ws2, r(bs2),
                           E=E, BN=1000)
    return out_e, out_v, out_u


# speedup vs baseline: 1.1571x; 1.1571x over previous
import functools

import jax
import jax.numpy as jnp
from jax import lax
from jax.experimental import pallas as pl
from jax.experimental.pallas import tpu as pltpu
from jax.experimental.pallas import tpu_sc as plsc

F32 = jnp.float32
NC = 2
NS = 16
NW = NC * NS


def _prep_body(nf, st, W_v, b_v, W_u, b_u, We1a, We1b, We1d, be1,
               v_o, A_o, B_o, u_o, c_o):
    v = jnp.maximum(jnp.dot(nf[...], W_v[...],
                            preferred_element_type=F32) + b_v[...], 0.0)
    v_o[...] = v
    A_o[...] = jnp.dot(v, We1a[...], preferred_element_type=F32)
    B_o[...] = jnp.dot(v, We1b[...], preferred_element_type=F32)
    u = jnp.maximum(jnp.dot(st[...], W_u[...],
                            preferred_element_type=F32) + b_u[...], 0.0)
    u_o[...] = u
    c_o[...] = jnp.dot(u, We1d[...], preferred_element_type=F32) + be1[...]


def _prep(nf, st, W_v, b_v, W_u, b_u, We1a, We1b, We1d, be1, BN):
    N, D = nf.shape
    grid = (N // BN,)
    row = lambda i: (i, 0)
    fix = lambda i: (0, 0)
    full = lambda i: (0, 0)
    return pl.pallas_call(
        _prep_body,
        grid=grid,
        in_specs=[
            pl.BlockSpec((BN, D), row),
            pl.BlockSpec((1, D), fix),
            pl.BlockSpec((D, D), full),
            pl.BlockSpec((1, D), fix),
            pl.BlockSpec((D, D), full),
            pl.BlockSpec((1, D), fix),
            pl.BlockSpec((D, D), full),
            pl.BlockSpec((D, D), full),
            pl.BlockSpec((D, D), full),
            pl.BlockSpec((1, D), fix),
        ],
        out_specs=[
            pl.BlockSpec((BN, D), row),
            pl.BlockSpec((BN, D), row),
            pl.BlockSpec((BN, D), row),
            pl.BlockSpec((1, D), fix),
            pl.BlockSpec((1, D), fix),
        ],
        out_shape=[
            jax.ShapeDtypeStruct((N, D), F32),
            jax.ShapeDtypeStruct((N, D), F32),
            jax.ShapeDtypeStruct((N, D), F32),
            jax.ShapeDtypeStruct((1, D), F32),
            jax.ShapeDtypeStruct((1, D), F32),
        ],
    )(nf, st, W_v, b_v, W_u, b_u, We1a, We1b, We1d, be1)


def _sc_gather(A, B, src, dst, CHUNK=128):
    N, D = A.shape
    E = src.shape[0]
    EPW = E // NW
    n_full = EPW // CHUNK
    tail = EPW - n_full * CHUNK
    mesh = plsc.VectorSubcoreMesh(core_axis_name="c", subcore_axis_name="s")

    def body(A_hbm, B_hbm, src_hbm, dst_hbm, G_hbm,
             idx_s, idx_d, bufA, bufB, idx_st, idx_dt, bufAt, bufBt, sem):
        wid = lax.axis_index("s") * NC + lax.axis_index("c")
        base_w = wid * EPW

        def do_chunk(base, n, i_s, i_d, bA, bB):
            cpI = pltpu.async_copy(src_hbm.at[pl.ds(base, n)], i_s, sem)
            cpJ = pltpu.async_copy(dst_hbm.at[pl.ds(base, n)], i_d, sem)
            cpI.wait()
            cpJ.wait()
            cpA = pltpu.async_copy(A_hbm.at[i_s], bA, sem)
            cpB = pltpu.async_copy(B_hbm.at[i_d], bB, sem)
            cpA.wait()
            cpB.wait()

            def add_row(r, _):
                for l in range(D // 16):
                    sl = pl.ds(l * 16, 16)
                    bA[r, sl] = bA[r, sl] + bB[r, sl]
                return 0

            lax.fori_loop(0, n, add_row, 0)
            pltpu.sync_copy(bA, G_hbm.at[pl.ds(base, n)])

        def chunk_loop(j, _):
            do_chunk(base_w + j * CHUNK, CHUNK, idx_s, idx_d, bufA, bufB)
            return 0

        lax.fori_loop(0, n_full, chunk_loop, 0)
        if tail:
            do_chunk(base_w + n_full * CHUNK, tail, idx_st, idx_dt, bufAt, bufBt)

    run = pl.kernel(
        body,
        out_type=jax.ShapeDtypeStruct((E, D), F32),
        mesh=mesh,
        scratch_types=[
            pltpu.VMEM((CHUNK,), jnp.int32),
            pltpu.VMEM((CHUNK,), jnp.int32),
            pltpu.VMEM((CHUNK, D), F32),
            pltpu.VMEM((CHUNK, D), F32),
            pltpu.VMEM((max(tail, 8),), jnp.int32),
            pltpu.VMEM((max(tail, 8),), jnp.int32),
            pltpu.VMEM((max(tail, 8), D), F32),
            pltpu.VMEM((max(tail, 8), D), F32),
            pltpu.SemaphoreType.DMA,
        ],
    )
    return run(A, B, src, dst)


def _edge_body(e0, G, W_e, b_e, We1c, c_row, We2, be2,
               oe_o, en_o, esum_o):
    i = pl.program_id(0)
    e = jnp.maximum(jnp.dot(e0[...], W_e[...],
                            preferred_element_type=F32) + b_e[...], 0.0)
    t = jnp.dot(e, We1c[...], preferred_element_type=F32) + G[...] + c_row[...]
    h = jnp.maximum(t, 0.0)
    en = jnp.maximum(jnp.dot(h, We2[...],
                             preferred_element_type=F32) + be2[...], 0.0)
    en_o[...] = en
    oe_o[...] = en + e0[...]

    @pl.when(i == 0)
    def _():
        esum_o[...] = jnp.zeros_like(esum_o)

    esum_o[...] += jnp.sum(en, axis=0, keepdims=True)


def _edge_pass(e0, G, W_e, b_e, We1c, c_row, We2, be2, BE):
    E, D = e0.shape
    grid = (E // BE,)
    row = lambda i: (i, 0)
    fix = lambda i: (0, 0)
    return pl.pallas_call(
        _edge_body,
        grid=grid,
        in_specs=[
            pl.BlockSpec((BE, D), row),
            pl.BlockSpec((BE, D), row),
            pl.BlockSpec((D, D), fix),
            pl.BlockSpec((1, D), fix),
            pl.BlockSpec((D, D), fix),
            pl.BlockSpec((1, D), fix),
            pl.BlockSpec((D, D), fix),
            pl.BlockSpec((1, D), fix),
        ],
        out_specs=[
            pl.BlockSpec((BE, D), row),
            pl.BlockSpec((BE, D), row),
            pl.BlockSpec((1, D), fix),
        ],
        out_shape=[
            jax.ShapeDtypeStruct((E, D), F32),
            jax.ShapeDtypeStruct((E, D), F32),
            jax.ShapeDtypeStruct((1, D), F32),
        ],
    )(e0, G, W_e, b_e, We1c, c_row, We2, be2)


def _sc_common(NP, CHUNK):
    NPS = NP // NS
    n_init = NPS // CHUNK
    init_tail = NPS - n_init * CHUNK
    return NPS, n_init, init_tail


def _mk_helpers(D, CHUNK, NPS, n_init, init_tail, sid, cid, NP):
    def fill(b, val):
        def fr(i, _):
            for l in range(D // 16):
                b[i, pl.ds(l * 16, 16)] = jnp.full((16,), val, F32)
            return 0
        lax.fori_loop(0, CHUNK, fr, 0)

    off0 = sid * NPS + n_init * CHUNK

    def zero_spmem(shS, buf2):
        fill(buf2, 0.0)

        def init_blk(k, _):
            pltpu.sync_copy(buf2,
                            shS.at[pl.ds(sid * NPS + k * CHUNK, CHUNK)])
            return 0
        lax.fori_loop(0, n_init, init_blk, 0)
        if init_tail:
            pltpu.sync_copy(buf2.at[pl.ds(0, init_tail)],
                            shS.at[pl.ds(off0, init_tail)])

    def writeout(shS, buf2, out_ref):
        def out_blk(k, _):
            off = sid * NPS + k * CHUNK
            pltpu.sync_copy(shS.at[pl.ds(off, CHUNK)], buf2)
            pltpu.sync_copy(buf2, out_ref.at[pl.ds(cid * NP + off, CHUNK)])
            return 0
        lax.fori_loop(0, n_init, out_blk, 0)
        if init_tail:
            pltpu.sync_copy(shS.at[pl.ds(off0, init_tail)],
                            buf2.at[pl.ds(0, init_tail)])
            pltpu.sync_copy(buf2.at[pl.ds(0, init_tail)],
                            out_ref.at[pl.ds(cid * NP + off0, init_tail)])

    return fill, zero_spmem, writeout


def _sc_sums(e_new, dst, NP, CHUNK=80):
    E, D = e_new.shape
    EPW = E // NW
    n_full = EPW // CHUNK
    tail = EPW - n_full * CHUNK
    TB = max(tail, 8)
    NPS, n_init, init_tail = _sc_common(NP, CHUNK)
    mesh = plsc.VectorSubcoreMesh(core_axis_name="c", subcore_axis_name="s")

    def body(en_hbm, dst_hbm, S_hbm, shS, buf, buf2, idx, buf_t, idx_t, sem):
        cid = lax.axis_index("c")
        sid = lax.axis_index("s")
        wid = sid * NC + cid
        base_w = wid * EPW
        fill, zero_spmem, writeout = _mk_helpers(
            D, CHUNK, NPS, n_init, init_tail, sid, cid, NP)

        zero_spmem(shS, buf2)
        plsc.subcore_barrier()

        def chunkA(j, _):
            base = base_w + j * CHUNK
            pltpu.sync_copy(dst_hbm.at[pl.ds(base, CHUNK)], idx)
            pltpu.sync_copy(en_hbm.at[pl.ds(base, CHUNK)], buf)
            pltpu.sync_copy(buf, shS.at[idx], add=True)
            return 0
        lax.fori_loop(0, n_full, chunkA, 0)
        if tail:
            base = base_w + n_full * CHUNK
            pltpu.sync_copy(dst_hbm.at[pl.ds(base, tail)], idx_t)
            pltpu.sync_copy(en_hbm.at[pl.ds(base, tail)], buf_t)
            pltpu.sync_copy(buf_t, shS.at[idx_t], add=True)
        plsc.subcore_barrier()
        writeout(shS, buf2, S_hbm)

    run = pl.kernel(
        body,
        out_type=jax.ShapeDtypeStruct((NC * NP, D), F32),
        mesh=mesh,
        scratch_types=[
            pltpu.VMEM_SHARED((NP, D), F32),
            pltpu.VMEM((CHUNK, D), F32),
            pltpu.VMEM((CHUNK, D), F32),
            pltpu.VMEM((CHUNK,), jnp.int32),
            pltpu.VMEM((TB, D), F32),
            pltpu.VMEM((TB,), jnp.int32),
            pltpu.SemaphoreType.DMA,
        ],
    )
    return run(e_new, dst)


def _sc_counts(dst, E, NP, CHUNK=80):
    D = 128
    EPW = E // NW
    n_full = EPW // CHUNK
    NPS, n_init, init_tail = _sc_common(NP, CHUNK)
    mesh = plsc.VectorSubcoreMesh(core_axis_name="c", subcore_axis_name="s")

    def body(dst_hbm, C_hbm, shS, buf, buf2, idx, sem):
        cid = lax.axis_index("c")
        sid = lax.axis_index("s")
        wid = sid * NC + cid
        base_w = wid * EPW
        fill, zero_spmem, writeout = _mk_helpers(
            D, CHUNK, NPS, n_init, init_tail, sid, cid, NP)

        zero_spmem(shS, buf2)
        fill(buf, 1.0)
        plsc.subcore_barrier()

        def chunkB(j, _):
            base = base_w + j * CHUNK
            pltpu.sync_copy(dst_hbm.at[pl.ds(base, CHUNK)], idx)
            pltpu.sync_copy(buf, shS.at[idx], add=True)
            return 0
        lax.fori_loop(0, n_full, chunkB, 0)
        plsc.subcore_barrier()
        writeout(shS, buf2, C_hbm)

    run = pl.kernel(
        body,
        out_type=jax.ShapeDtypeStruct((NC * NP, D), F32),
        mesh=mesh,
        scratch_types=[
            pltpu.VMEM_SHARED((NP, D), F32),
            pltpu.VMEM((CHUNK, D), F32),
            pltpu.VMEM((CHUNK, D), F32),
            pltpu.VMEM((CHUNK,), jnp.int32),
            pltpu.SemaphoreType.DMA,
        ],
    )
    return run(dst)


def _finish_body(S, C16, v, nf, u, st, esum,
                 Wn1a, Wn1b, Wn1c, bn1, Wn2, bn2,
                 Ws1a, Ws1b, Ws1c, bs1, Ws2, bs2,
                 inv_e, inv_n,
                 ov_o, ou_o, vsum):
    i = pl.program_id(0)
    n_blocks = pl.num_programs(0)
    s = jnp.sum(S[...], axis=0)
    cnt2 = C16[0] + C16[1]
    cnt = cnt2[:, 0:1]
    ve = s / jnp.maximum(cnt, 1.0)
    pre = (jnp.dot(v[...], Wn1a[...], preferred_element_type=F32)
           + jnp.dot(ve, Wn1b[...], preferred_element_type=F32)
           + jnp.dot(u[...], Wn1c[...], preferred_element_type=F32)
           + bn1[...])
    h = jnp.maximum(pre, 0.0)
    vn = jnp.maximum(jnp.dot(h, Wn2[...],
                             preferred_element_type=F32) + bn2[...], 0.0)
    ov_o[...] = vn + nf[...]

    @pl.when(i == 0)
    def _():
        vsum[...] = jnp.zeros_like(vsum)

    vsum[...] += jnp.sum(vn, axis=0, keepdims=True)

    @pl.when(i == n_blocks - 1)
    def _():
        u_edge = esum[...] * inv_e[0]
        u_vertex = vsum[...] * inv_n[0]
        spre = (jnp.dot(u[...], Ws1a[...], preferred_element_type=F32)
                + jnp.dot(u_edge, Ws1b[...], preferred_element_type=F32)
                + jnp.dot(u_vertex, Ws1c[...], preferred_element_type=F32)
                + bs1[...])
        h2 = jnp.maximum(spre, 0.0)
        un = jnp.maximum(jnp.dot(h2, Ws2[...],
                                 preferred_element_type=F32) + bs2[...], 0.0)
        ou_o[...] = un + st[...]


def _finish(S, C16, v, nf, u, st, esum,
            Wn1a, Wn1b, Wn1c, bn1, Wn2, bn2,
            Ws1a, Ws1b, Ws1c, bs1, Ws2, bs2, E, BN):
    N, D = nf.shape
    KS = S.shape[0]
    CW = C16.shape[-1]
    grid = (N // BN,)
    row = lambda i: (i, 0)
    row3 = lambda i: (0, i, 0)
    fix = lambda i: (0, 0)
    inv_e = jnp.full((1,), 1.0 / E, F32)
    inv_n = jnp.full((1,), 1.0 / N, F32)
    return pl.pallas_call(
        _finish_body,
        grid=grid,
        in_specs=[
            pl.BlockSpec((KS, BN, D), row3),
            pl.BlockSpec((2, BN, CW), row3),
            pl.BlockSpec((BN, D), row),
            pl.BlockSpec((BN, D), row),
            pl.BlockSpec((1, D), fix),
            pl.BlockSpec((1, D), fix),
            pl.BlockSpec((1, D), fix),
            pl.BlockSpec((D, D), fix),
            pl.BlockSpec((D, D), fix),
            pl.BlockSpec((D, D), fix),
            pl.BlockSpec((1, D), fix),
            pl.BlockSpec((D, D), fix),
            pl.BlockSpec((1, D), fix),
            pl.BlockSpec((D, D), fix),
            pl.BlockSpec((D, D), fix),
            pl.BlockSpec((D, D), fix),
            pl.BlockSpec((1, D), fix),
            pl.BlockSpec((D, D), fix),
            pl.BlockSpec((1, D), fix),
            pl.BlockSpec(memory_space=pltpu.SMEM),
            pl.BlockSpec(memory_space=pltpu.SMEM),
        ],
        out_specs=[
            pl.BlockSpec((BN, D), row),
            pl.BlockSpec((1, D), fix),
        ],
        out_shape=[
            jax.ShapeDtypeStruct((N, D), F32),
            jax.ShapeDtypeStruct((1, D), F32),
        ],
        scratch_shapes=[pltpu.VMEM((1, D), F32)],
    )(S, C16, v, nf, u, st, esum,
      Wn1a, Wn1b, Wn1c, bn1, Wn2, bn2,
      Ws1a, Ws1b, Ws1c, bs1, Ws2, bs2, inv_e, inv_n)


def kernel(edge_feat, node_feat, state_attr, edge_index,
           W_e, b_e, W_v, b_v, W_u, b_u,
           We1, be1, We2, be2,
           Wn1, bn1, Wn2, bn2,
           Ws1, bs1, Ws2, bs2):
    E, D = edge_feat.shape
    N = node_feat.shape[0]
    src = edge_index[0]
    dst = edge_index[1]

    We1a, We1b = We1[0:D], We1[D:2 * D]
    We1c, We1d = We1[2 * D:3 * D], We1[3 * D:4 * D]
    Wn1a, Wn1b, Wn1c = Wn1[0:D], Wn1[D:2 * D], Wn1[2 * D:3 * D]
    Ws1a, Ws1b, Ws1c = Ws1[0:D], Ws1[D:2 * D], Ws1[2 * D:3 * D]
    r = lambda x: x.reshape(1, D)

    v, A, B, u, c_row = _prep(node_feat, state_attr, W_v, r(b_v), W_u, r(b_u),
                              We1a, We1b, We1d, r(be1), BN=1000)
    NP = ((N + NS * 8 - 1) // (NS * 8)) * (NS * 8)
    G = _sc_gather(A, B, src, dst)
    C = _sc_counts(dst, E, NP)
    out_e, e_new, e_sum = _edge_pass(edge_feat, G, W_e, r(b_e), We1c,
                                     c_row, We2, r(be2), BE=4000)
    S = _sc_sums(e_new, dst, NP)
    S = S.reshape(NC, NP, D)
    C16 = C.reshape(NC, NP, D)
    out_v, out_u = _finish(S, C16, v, node_feat, u, state_attr, e_sum,
                           Wn1a, Wn1b, Wn1c, r(bn1), Wn2, r(bn2),
                           Ws1a, Ws1b, Ws1c, r(bs1), Ws2, r(bs2),
                           E=E, BN=1000)
    return out_e, out_v, out_u


# speedup vs baseline: 1.2288x; 1.0619x over previous
import functools

import jax
import jax.numpy as jnp
from jax import lax
from jax.experimental import pallas as pl
from jax.experimental.pallas import tpu as pltpu
from jax.experimental.pallas import tpu_sc as plsc

F32 = jnp.float32
NC = 2
NS = 16
NW = NC * NS


def _prep_body(nf, st, W_v, b_v, W_u, b_u, We1a, We1b, We1d, be1,
               v_o, A_o, B_o, u_o, c_o):
    v = jnp.maximum(jnp.dot(nf[...], W_v[...],
                            preferred_element_type=F32) + b_v[...], 0.0)
    v_o[...] = v
    A_o[...] = jnp.dot(v, We1a[...], preferred_element_type=F32)
    B_o[...] = jnp.dot(v, We1b[...], preferred_element_type=F32)
    u = jnp.maximum(jnp.dot(st[...], W_u[...],
                            preferred_element_type=F32) + b_u[...], 0.0)
    u_o[...] = u
    c_o[...] = jnp.dot(u, We1d[...], preferred_element_type=F32) + be1[...]


def _prep(nf, st, W_v, b_v, W_u, b_u, We1a, We1b, We1d, be1, BN):
    N, D = nf.shape
    grid = (N // BN,)
    row = lambda i: (i, 0)
    fix = lambda i: (0, 0)
    full = lambda i: (0, 0)
    return pl.pallas_call(
        _prep_body,
        grid=grid,
        in_specs=[
            pl.BlockSpec((BN, D), row),
            pl.BlockSpec((1, D), fix),
            pl.BlockSpec((D, D), full),
            pl.BlockSpec((1, D), fix),
            pl.BlockSpec((D, D), full),
            pl.BlockSpec((1, D), fix),
            pl.BlockSpec((D, D), full),
            pl.BlockSpec((D, D), full),
            pl.BlockSpec((D, D), full),
            pl.BlockSpec((1, D), fix),
        ],
        out_specs=[
            pl.BlockSpec((BN, D), row),
            pl.BlockSpec((BN, D), row),
            pl.BlockSpec((BN, D), row),
            pl.BlockSpec((1, D), fix),
            pl.BlockSpec((1, D), fix),
        ],
        out_shape=[
            jax.ShapeDtypeStruct((N, D), F32),
            jax.ShapeDtypeStruct((N, D), F32),
            jax.ShapeDtypeStruct((N, D), F32),
            jax.ShapeDtypeStruct((1, D), F32),
            jax.ShapeDtypeStruct((1, D), F32),
        ],
    )(nf, st, W_v, b_v, W_u, b_u, We1a, We1b, We1d, be1)


def _sc_gather(A, B, src, dst, CHUNK=128):
    N, D = A.shape
    E = src.shape[0]
    EPW = E // NW
    n_full = EPW // CHUNK
    tail = EPW - n_full * CHUNK
    mesh = plsc.VectorSubcoreMesh(core_axis_name="c", subcore_axis_name="s")

    def body(A_hbm, B_hbm, src_hbm, dst_hbm, G_hbm,
             idx_s, idx_d, bufA, bufB, idx_st, idx_dt, bufAt, bufBt, sem):
        wid = lax.axis_index("s") * NC + lax.axis_index("c")
        base_w = wid * EPW

        def do_chunk(base, n, i_s, i_d, bA, bB):
            cpI = pltpu.async_copy(src_hbm.at[pl.ds(base, n)], i_s, sem)
            cpJ = pltpu.async_copy(dst_hbm.at[pl.ds(base, n)], i_d, sem)
            cpI.wait()
            cpJ.wait()
            cpA = pltpu.async_copy(A_hbm.at[i_s], bA, sem)
            cpB = pltpu.async_copy(B_hbm.at[i_d], bB, sem)
            cpA.wait()
            cpB.wait()

            def add_row(r, _):
                for l in range(D // 16):
                    sl = pl.ds(l * 16, 16)
                    bA[r, sl] = bA[r, sl] + bB[r, sl]
                return 0

            lax.fori_loop(0, n, add_row, 0)
            pltpu.sync_copy(bA, G_hbm.at[pl.ds(base, n)])

        def chunk_loop(j, _):
            do_chunk(base_w + j * CHUNK, CHUNK, idx_s, idx_d, bufA, bufB)
            return 0

        lax.fori_loop(0, n_full, chunk_loop, 0)
        if tail:
            do_chunk(base_w + n_full * CHUNK, tail, idx_st, idx_dt, bufAt, bufBt)

    run = pl.kernel(
        body,
        out_type=jax.ShapeDtypeStruct((E, D), F32),
        mesh=mesh,
        scratch_types=[
            pltpu.VMEM((CHUNK,), jnp.int32),
            pltpu.VMEM((CHUNK,), jnp.int32),
            pltpu.VMEM((CHUNK, D), F32),
            pltpu.VMEM((CHUNK, D), F32),
            pltpu.VMEM((max(tail, 8),), jnp.int32),
            pltpu.VMEM((max(tail, 8),), jnp.int32),
            pltpu.VMEM((max(tail, 8), D), F32),
            pltpu.VMEM((max(tail, 8), D), F32),
            pltpu.SemaphoreType.DMA,
        ],
    )
    return run(A, B, src, dst)


def _edge_body(e0, G, W_e, b_e, We1c, c_row, We2, be2,
               oe_o, en_o, esum_o):
    i = pl.program_id(0)
    e = jnp.maximum(jnp.dot(e0[...], W_e[...],
                            preferred_element_type=F32) + b_e[...], 0.0)
    t = jnp.dot(e, We1c[...], preferred_element_type=F32) + G[...] + c_row[...]
    h = jnp.maximum(t, 0.0)
    en = jnp.maximum(jnp.dot(h, We2[...],
                             preferred_element_type=F32) + be2[...], 0.0)
    en_o[...] = en
    oe_o[...] = en + e0[...]

    @pl.when(i == 0)
    def _():
        esum_o[...] = jnp.zeros_like(esum_o)

    esum_o[...] += jnp.sum(en, axis=0, keepdims=True)


def _edge_pass(e0, G, W_e, b_e, We1c, c_row, We2, be2, BE):
    E, D = e0.shape
    grid = (E // BE,)
    row = lambda i: (i, 0)
    fix = lambda i: (0, 0)
    return pl.pallas_call(
        _edge_body,
        grid=grid,
        in_specs=[
            pl.BlockSpec((BE, D), row),
            pl.BlockSpec((BE, D), row),
            pl.BlockSpec((D, D), fix),
            pl.BlockSpec((1, D), fix),
            pl.BlockSpec((D, D), fix),
            pl.BlockSpec((1, D), fix),
            pl.BlockSpec((D, D), fix),
            pl.BlockSpec((1, D), fix),
        ],
        out_specs=[
            pl.BlockSpec((BE, D), row),
            pl.BlockSpec((BE, D), row),
            pl.BlockSpec((1, D), fix),
        ],
        out_shape=[
            jax.ShapeDtypeStruct((E, D), F32),
            jax.ShapeDtypeStruct((E, D), F32),
            jax.ShapeDtypeStruct((1, D), F32),
        ],
    )(e0, G, W_e, b_e, We1c, c_row, We2, be2)


def _sc_common(NP, CHUNK):
    NPS = NP // NS
    n_init = NPS // CHUNK
    init_tail = NPS - n_init * CHUNK
    return NPS, n_init, init_tail


def _mk_helpers(D, CHUNK, NPS, n_init, init_tail, sid, cid, NP):
    def fill(b, val):
        def fr(i, _):
            for l in range(D // 16):
                b[i, pl.ds(l * 16, 16)] = jnp.full((16,), val, F32)
            return 0
        lax.fori_loop(0, CHUNK, fr, 0)

    off0 = sid * NPS + n_init * CHUNK

    def zero_spmem(shS, buf2):
        fill(buf2, 0.0)

        def init_blk(k, _):
            pltpu.sync_copy(buf2,
                            shS.at[pl.ds(sid * NPS + k * CHUNK, CHUNK)])
            return 0
        lax.fori_loop(0, n_init, init_blk, 0)
        if init_tail:
            pltpu.sync_copy(buf2.at[pl.ds(0, init_tail)],
                            shS.at[pl.ds(off0, init_tail)])

    def writeout(shS, buf2, out_ref):
        def out_blk(k, _):
            off = sid * NPS + k * CHUNK
            pltpu.sync_copy(shS.at[pl.ds(off, CHUNK)], buf2)
            pltpu.sync_copy(buf2, out_ref.at[pl.ds(cid * NP + off, CHUNK)])
            return 0
        lax.fori_loop(0, n_init, out_blk, 0)
        if init_tail:
            pltpu.sync_copy(shS.at[pl.ds(off0, init_tail)],
                            buf2.at[pl.ds(0, init_tail)])
            pltpu.sync_copy(buf2.at[pl.ds(0, init_tail)],
                            out_ref.at[pl.ds(cid * NP + off0, init_tail)])

    return fill, zero_spmem, writeout


def _sc_sums(e_new, dst, NP, CHUNK=80):
    E, D = e_new.shape
    EPW = E // NW
    n_full = EPW // CHUNK
    tail = EPW - n_full * CHUNK
    TB = max(tail, 8)
    NPS, n_init, init_tail = _sc_common(NP, CHUNK)
    mesh = plsc.VectorSubcoreMesh(core_axis_name="c", subcore_axis_name="s")

    def body(en_hbm, dst_hbm, S_hbm, shS, buf, buf2, idx, buf_t, idx_t, sem):
        cid = lax.axis_index("c")
        sid = lax.axis_index("s")
        wid = sid * NC + cid
        base_w = wid * EPW
        fill, zero_spmem, writeout = _mk_helpers(
            D, CHUNK, NPS, n_init, init_tail, sid, cid, NP)

        zero_spmem(shS, buf2)
        plsc.subcore_barrier()

        def chunkA(j, _):
            base = base_w + j * CHUNK
            cpI = pltpu.async_copy(dst_hbm.at[pl.ds(base, CHUNK)], idx, sem)
            cpB = pltpu.async_copy(en_hbm.at[pl.ds(base, CHUNK)], buf, sem)
            cpI.wait()
            cpB.wait()
            pltpu.sync_copy(buf, shS.at[idx], add=True)
            return 0
        lax.fori_loop(0, n_full, chunkA, 0)
        if tail:
            base = base_w + n_full * CHUNK
            cpI = pltpu.async_copy(dst_hbm.at[pl.ds(base, tail)], idx_t, sem)
            cpB = pltpu.async_copy(en_hbm.at[pl.ds(base, tail)], buf_t, sem)
            cpI.wait()
            cpB.wait()
            pltpu.sync_copy(buf_t, shS.at[idx_t], add=True)
        plsc.subcore_barrier()
        writeout(shS, buf2, S_hbm)

    run = pl.kernel(
        body,
        out_type=jax.ShapeDtypeStruct((NC * NP, D), F32),
        mesh=mesh,
        scratch_types=[
            pltpu.VMEM_SHARED((NP, D), F32),
            pltpu.VMEM((CHUNK, D), F32),
            pltpu.VMEM((CHUNK, D), F32),
            pltpu.VMEM((CHUNK,), jnp.int32),
            pltpu.VMEM((TB, D), F32),
            pltpu.VMEM((TB,), jnp.int32),
            pltpu.SemaphoreType.DMA,
        ],
    )
    return run(e_new, dst)


def _sc_counts(dst, E, NP, CHUNK=80):
    D = 128
    EPW = E // NW
    n_full = EPW // CHUNK
    NPS, n_init, init_tail = _sc_common(NP, CHUNK)
    mesh = plsc.VectorSubcoreMesh(core_axis_name="c", subcore_axis_name="s")

    def body(dst_hbm, C_hbm, shS, buf, buf2, idx, sem):
        cid = lax.axis_index("c")
        sid = lax.axis_index("s")
        wid = sid * NC + cid
        base_w = wid * EPW
        fill, zero_spmem, writeout = _mk_helpers(
            D, CHUNK, NPS, n_init, init_tail, sid, cid, NP)

        zero_spmem(shS, buf2)
        fill(buf, 1.0)
        plsc.subcore_barrier()

        def chunkB(j, _):
            base = base_w + j * CHUNK
            pltpu.sync_copy(dst_hbm.at[pl.ds(base, CHUNK)], idx)
            pltpu.sync_copy(buf, shS.at[idx], add=True)
            return 0
        lax.fori_loop(0, n_full, chunkB, 0)
        plsc.subcore_barrier()
        writeout(shS, buf2, C_hbm)

    run = pl.kernel(
        body,
        out_type=jax.ShapeDtypeStruct((NC * NP, D), F32),
        mesh=mesh,
        scratch_types=[
            pltpu.VMEM_SHARED((NP, D), F32),
            pltpu.VMEM((CHUNK, D), F32),
            pltpu.VMEM((CHUNK, D), F32),
            pltpu.VMEM((CHUNK,), jnp.int32),
            pltpu.SemaphoreType.DMA,
        ],
    )
    return run(dst)


def _finish_body(S, C16, v, nf, u, st, esum,
                 Wn1a, Wn1b, Wn1c, bn1, Wn2, bn2,
                 Ws1a, Ws1b, Ws1c, bs1, Ws2, bs2,
                 inv_e, inv_n,
                 ov_o, ou_o, vsum):
    i = pl.program_id(0)
    n_blocks = pl.num_programs(0)
    s = jnp.sum(S[...], axis=0)
    cnt2 = C16[0] + C16[1]
    cnt = cnt2[:, 0:1]
    ve = s / jnp.maximum(cnt, 1.0)
    pre = (jnp.dot(v[...], Wn1a[...], preferred_element_type=F32)
           + jnp.dot(ve, Wn1b[...], preferred_element_type=F32)
           + jnp.dot(u[...], Wn1c[...], preferred_element_type=F32)
           + bn1[...])
    h = jnp.maximum(pre, 0.0)
    vn = jnp.maximum(jnp.dot(h, Wn2[...],
                             preferred_element_type=F32) + bn2[...], 0.0)
    ov_o[...] = vn + nf[...]

    @pl.when(i == 0)
    def _():
        vsum[...] = jnp.zeros_like(vsum)

    vsum[...] += jnp.sum(vn, axis=0, keepdims=True)

    @pl.when(i == n_blocks - 1)
    def _():
        u_edge = esum[...] * inv_e[0]
        u_vertex = vsum[...] * inv_n[0]
        spre = (jnp.dot(u[...], Ws1a[...], preferred_element_type=F32)
                + jnp.dot(u_edge, Ws1b[...], preferred_element_type=F32)
                + jnp.dot(u_vertex, Ws1c[...], preferred_element_type=F32)
                + bs1[...])
        h2 = jnp.maximum(spre, 0.0)
        un = jnp.maximum(jnp.dot(h2, Ws2[...],
                                 preferred_element_type=F32) + bs2[...], 0.0)
        ou_o[...] = un + st[...]


def _finish(S, C16, v, nf, u, st, esum,
            Wn1a, Wn1b, Wn1c, bn1, Wn2, bn2,
            Ws1a, Ws1b, Ws1c, bs1, Ws2, bs2, E, BN):
    N, D = nf.shape
    KS = S.shape[0]
    CW = C16.shape[-1]
    grid = (N // BN,)
    row = lambda i: (i, 0)
    row3 = lambda i: (0, i, 0)
    fix = lambda i: (0, 0)
    inv_e = jnp.full((1,), 1.0 / E, F32)
    inv_n = jnp.full((1,), 1.0 / N, F32)
    return pl.pallas_call(
        _finish_body,
        grid=grid,
        in_specs=[
            pl.BlockSpec((KS, BN, D), row3),
            pl.BlockSpec((2, BN, CW), row3),
            pl.BlockSpec((BN, D), row),
            pl.BlockSpec((BN, D), row),
            pl.BlockSpec((1, D), fix),
            pl.BlockSpec((1, D), fix),
            pl.BlockSpec((1, D), fix),
            pl.BlockSpec((D, D), fix),
            pl.BlockSpec((D, D), fix),
            pl.BlockSpec((D, D), fix),
            pl.BlockSpec((1, D), fix),
            pl.BlockSpec((D, D), fix),
            pl.BlockSpec((1, D), fix),
            pl.BlockSpec((D, D), fix),
            pl.BlockSpec((D, D), fix),
            pl.BlockSpec((D, D), fix),
            pl.BlockSpec((1, D), fix),
            pl.BlockSpec((D, D), fix),
            pl.BlockSpec((1, D), fix),
            pl.BlockSpec(memory_space=pltpu.SMEM),
            pl.BlockSpec(memory_space=pltpu.SMEM),
        ],
        out_specs=[
            pl.BlockSpec((BN, D), row),
            pl.BlockSpec((1, D), fix),
        ],
        out_shape=[
            jax.ShapeDtypeStruct((N, D), F32),
            jax.ShapeDtypeStruct((1, D), F32),
        ],
        scratch_shapes=[pltpu.VMEM((1, D), F32)],
    )(S, C16, v, nf, u, st, esum,
      Wn1a, Wn1b, Wn1c, bn1, Wn2, bn2,
      Ws1a, Ws1b, Ws1c, bs1, Ws2, bs2, inv_e, inv_n)


def kernel(edge_feat, node_feat, state_attr, edge_index,
           W_e, b_e, W_v, b_v, W_u, b_u,
           We1, be1, We2, be2,
           Wn1, bn1, Wn2, bn2,
           Ws1, bs1, Ws2, bs2):
    E, D = edge_feat.shape
    N = node_feat.shape[0]
    src = edge_index[0]
    dst = edge_index[1]

    We1a, We1b = We1[0:D], We1[D:2 * D]
    We1c, We1d = We1[2 * D:3 * D], We1[3 * D:4 * D]
    Wn1a, Wn1b, Wn1c = Wn1[0:D], Wn1[D:2 * D], Wn1[2 * D:3 * D]
    Ws1a, Ws1b, Ws1c = Ws1[0:D], Ws1[D:2 * D], Ws1[2 * D:3 * D]
    r = lambda x: x.reshape(1, D)

    v, A, B, u, c_row = _prep(node_feat, state_attr, W_v, r(b_v), W_u, r(b_u),
                              We1a, We1b, We1d, r(be1), BN=1000)
    NP = ((N + NS * 8 - 1) // (NS * 8)) * (NS * 8)
    G = _sc_gather(A, B, src, dst)
    C = _sc_counts(dst, E, NP)
    out_e, e_new, e_sum = _edge_pass(edge_feat, G, W_e, r(b_e), We1c,
                                     c_row, We2, r(be2), BE=4000)
    S = _sc_sums(e_new, dst, NP)
    S = S.reshape(NC, NP, D)
    C16 = C.reshape(NC, NP, D)
    out_v, out_u = _finish(S, C16, v, node_feat, u, state_attr, e_sum,
                           Wn1a, Wn1b, Wn1c, r(bn1), Wn2, r(bn2),
                           Ws1a, Ws1b, Ws1c, r(bs1), Ws2, r(bs2),
                           E=E, BN=1000)
    return out_e, out_v, out_u
